# baseline (speedup 1.0000x reference)
import functools
import jax
import jax.numpy as jnp
from jax import lax
from jax.experimental import pallas as pl
from jax.experimental.pallas import tpu as pltpu
from jax.experimental.pallas import tpu_sc as plsc

N = 10000
E = 320000
D = 128
T = 4
BN = 2000
NB = N // BN

NC = 2
NS = 16
HN = N // 2
JBASE = HN
JMASK = 1023
AROWS = 6144
ARW = AROWS // NS
ECW = E // NS
C2 = 128
NCHP = 162
IDXROWS = 164


def _make_prop():
    mesh = plsc.VectorSubcoreMesh(core_axis_name="c", subcore_axis_name="s")

    @functools.partial(
        pl.kernel,
        out_type=jax.ShapeDtypeStruct((NC, NS, ARW, D), jnp.float32),
        mesh=mesh,
        scratch_types=[
            pltpu.VMEM((IDXROWS, C2), jnp.int32),
            pltpu.VMEM((IDXROWS, C2), jnp.int32),
            [pltpu.VMEM((C2, D), jnp.float32)] * 2,
            pltpu.VMEM_SHARED((AROWS, D), jnp.float32),
            [pltpu.SemaphoreType.DMA] * 2,
            [pltpu.SemaphoreType.DMA] * 2,
        ],
    )
    def prop(xs_hbm, srcm_hbm, dstm_hbm, zeros_hbm, out_hbm,
             src_v, dst_v, bufs, acc, gsem, ssem):
        c = lax.axis_index("c")
        s = lax.axis_index("s")
        pltpu.sync_copy(zeros_hbm, acc.at[pl.ds(s * ARW, ARW)])
        pltpu.sync_copy(srcm_hbm.at[s], src_v)
        pltpu.sync_copy(dstm_hbm.at[c, s], dst_v)
        plsc.subcore_barrier()

        def gath(j, k):
            pltpu.async_copy(xs_hbm.at[src_v.at[j]], bufs[k], gsem[k])

        def wait_g(j, k):
            pltpu.make_async_copy(xs_hbm.at[src_v.at[j]], bufs[k], gsem[k]).wait()

        def scat(j, k):
            pltpu.async_copy(bufs[k], acc.at[dst_v.at[j]], ssem[k], add=True)

        def wait_s(j, k):
            pltpu.make_async_copy(bufs[k], acc.at[dst_v.at[j]], ssem[k]).wait()

        gath(0, 0)
        gath(1, 1)
        wait_g(0, 0)
        scat(0, 0)
        wait_s(0, 0)
        gath(2, 0)
        wait_g(1, 1)
        scat(1, 1)

        def body(i, carry):
            j = 2 * i
            wait_s(j - 1, 1)
            gath(j + 1, 1)
            wait_g(j, 0)
            scat(j, 0)
            wait_s(j, 0)
            gath(j + 2, 0)
            wait_g(j + 1, 1)
            scat(j + 1, 1)
            return carry

        lax.fori_loop(1, NCHP // 2, body, 0)
        wait_s(161, 1)
        wait_g(162, 0)
        plsc.subcore_barrier()
        pltpu.sync_copy(acc.at[pl.ds(s * ARW, ARW)], out_hbm.at[c, s])

    return prop


_prop128 = _make_prop()


def _pad_idx(a, fill):
    pad = jnp.full((a.shape[0], IDXROWS * C2 - ECW), fill, jnp.int32)
    return jnp.concatenate([a, pad], axis=1).reshape(a.shape[0], IDXROWS, C2)


def _local_dst(d):
    j = JBASE + (d & JMASK)
    l0 = jnp.where(d < HN, d, j)
    l1 = jnp.where(d >= HN, d - HN, j)
    return l0, l1


def _prop_indices(idx):
    l0, l1 = _local_dst(idx)
    return jnp.stack([_pad_idx(l0.reshape(NS, ECW), JBASE),
                      _pad_idx(l1.reshape(NS, ECW), JBASE)])


def _merge_halves(out):
    halves = out.reshape(NC, AROWS, D)
    return jnp.concatenate([halves[0, :HN], halves[1, :HN]], axis=0)


def _prep_body(x_ref, d_ref, w0_ref, b_ref, xs_ref, xc_ref):
    x = x_ref[0]
    d = d_ref[...]
    xs_ref[0] = x * d
    xc_ref[0] = jnp.dot(x, w0_ref[...], preferred_element_type=jnp.float32) + b_ref[...]


def _tc_prep(x, dis2d, w0stack, bstack):
    return pl.pallas_call(
        _prep_body,
        grid=(T, NB),
        in_specs=[
            pl.BlockSpec((1, BN, D), lambda t, i: (t, i, 0)),
            pl.BlockSpec((BN, 1), lambda t, i: (i, 0)),
            pl.BlockSpec((D, 3 * D), lambda t, i: (0, 0)),
            pl.BlockSpec((1, 3 * D), lambda t, i: (0, 0)),
        ],
        out_specs=[
            pl.BlockSpec((1, BN, D), lambda t, i: (t, i, 0)),
            pl.BlockSpec((1, BN, 3 * D), lambda t, i: (t, i, 0)),
        ],
        out_shape=[
            jax.ShapeDtypeStruct((T, N, D), jnp.float32),
            jax.ShapeDtypeStruct((T, N, 3 * D), jnp.float32),
        ],
    )(x, dis2d, w0stack, bstack)


def _xmerge_body(p_ref, d_ref, xc_ref, w1_ref, out_ref):
    p = p_ref[0]
    d = d_ref[...]
    pxs = -(d * p)
    out_ref[0] = xc_ref[0] + jnp.dot(pxs, w1_ref[...], preferred_element_type=jnp.float32)


def _tc_xmerge(pxp, dis2d, xc, w1stack):
    return pl.pallas_call(
        _xmerge_body,
        grid=(T, NB),
        in_specs=[
            pl.BlockSpec((1, BN, D), lambda t, i: (t, i, 0)),
            pl.BlockSpec((BN, 1), lambda t, i: (i, 0)),
            pl.BlockSpec((1, BN, 3 * D), lambda t, i: (t, i, 0)),
            pl.BlockSpec((D, 3 * D), lambda t, i: (0, 0)),
        ],
        out_specs=pl.BlockSpec((1, BN, 3 * D), lambda t, i: (t, i, 0)),
        out_shape=jax.ShapeDtypeStruct((T, N, 3 * D), jnp.float32),
    )(pxp, dis2d, xc, w1stack)


def _step0_body(xc_ref, h_ref):
    c = xc_ref[...]
    z = jax.nn.sigmoid(c[:, :D])
    ht = jnp.tanh(c[:, 2 * D:])
    h_ref[...] = (1.0 - z) * ht


def _tc_step0(xc0):
    return pl.pallas_call(
        _step0_body,
        grid=(NB,),
        in_specs=[pl.BlockSpec((BN, 3 * D), lambda i: (i, 0))],
        out_specs=pl.BlockSpec((BN, D), lambda i: (i, 0)),
        out_shape=jax.ShapeDtypeStruct((N, D), jnp.float32),
    )(xc0)


def _preh_body(h_ref, d_ref, w_ref, hs_ref, hw_ref):
    h = h_ref[...]
    hs_ref[...] = h * d_ref[...]
    hw_ref[...] = jnp.dot(h, w_ref[...], preferred_element_type=jnp.float32)


def _tc_preh(h, dis2d, wh0zr):
    return pl.pallas_call(
        _preh_body,
        grid=(NB,),
        in_specs=[
            pl.BlockSpec((BN, D), lambda i: (i, 0)),
            pl.BlockSpec((BN, 1), lambda i: (i, 0)),
            pl.BlockSpec((D, 2 * D), lambda i: (0, 0)),
        ],
        out_specs=[
            pl.BlockSpec((BN, D), lambda i: (i, 0)),
            pl.BlockSpec((BN, 2 * D), lambda i: (i, 0)),
        ],
        out_shape=[
            jax.ShapeDtypeStruct((N, D), jnp.float32),
            jax.ShapeDtypeStruct((N, 2 * D), jnp.float32),
        ],
    )(h, dis2d, wh0zr)


def _gates_body(xczr_ref, hw_ref, php_ref, d_ref, w1_ref, h_ref, whh0_ref,
                z_ref, hrs_ref, hrw_ref):
    p = php_ref[...]
    d = d_ref[...]
    phs = -(d * p)
    g = xczr_ref[...] + hw_ref[...] + jnp.dot(phs, w1_ref[...],
                                              preferred_element_type=jnp.float32)
    z = jax.nn.sigmoid(g[:, :D])
    r = jax.nn.sigmoid(g[:, D:])
    hr = h_ref[...] * r
    z_ref[...] = z
    hrs_ref[...] = d * hr
    hrw_ref[...] = jnp.dot(hr, whh0_ref[...], preferred_element_type=jnp.float32)


def _tc_gates(xczr, hw0, php, dis2d, wh1zr, h, whh0):
    return pl.pallas_call(
        _gates_body,
        grid=(NB,),
        in_specs=[
            pl.BlockSpec((BN, 2 * D), lambda i: (i, 0)),
            pl.BlockSpec((BN, 2 * D), lambda i: (i, 0)),
            pl.BlockSpec((BN, D), lambda i: (i, 0)),
            pl.BlockSpec((BN, 1), lambda i: (i, 0)),
            pl.BlockSpec((D, 2 * D), lambda i: (0, 0)),
            pl.BlockSpec((BN, D), lambda i: (i, 0)),
            pl.BlockSpec((D, D), lambda i: (0, 0)),
        ],
        out_specs=[
            pl.BlockSpec((BN, D), lambda i: (i, 0)),
            pl.BlockSpec((BN, D), lambda i: (i, 0)),
            pl.BlockSpec((BN, D), lambda i: (i, 0)),
        ],
        out_shape=[
            jax.ShapeDtypeStruct((N, D), jnp.float32),
            jax.ShapeDtypeStruct((N, D), jnp.float32),
            jax.ShapeDtypeStruct((N, D), jnp.float32),
        ],
    )(xczr, hw0, php, dis2d, wh1zr, h, whh0)


def _newh_body(xch_ref, hrw_ref, pqp_ref, d_ref, whh1_ref, z_ref, h_ref, out_ref):
    q = -(d_ref[...] * pqp_ref[...])
    ht = jnp.tanh(xch_ref[...] + hrw_ref[...] +
                  jnp.dot(q, whh1_ref[...], preferred_element_type=jnp.float32))
    z = z_ref[...]
    out_ref[...] = z * h_ref[...] + (1.0 - z) * ht


def _tc_newh(xch, hrw0, pqp, dis2d, whh1, z, h):
    return pl.pallas_call(
        _newh_body,
        grid=(NB,),
        in_specs=[
            pl.BlockSpec((BN, D), lambda i: (i, 0)),
            pl.BlockSpec((BN, D), lambda i: (i, 0)),
            pl.BlockSpec((BN, D), lambda i: (i, 0)),
            pl.BlockSpec((BN, 1), lambda i: (i, 0)),
            pl.BlockSpec((D, D), lambda i: (0, 0)),
            pl.BlockSpec((BN, D), lambda i: (i, 0)),
            pl.BlockSpec((BN, D), lambda i: (i, 0)),
        ],
        out_specs=pl.BlockSpec((BN, D), lambda i: (i, 0)),
        out_shape=jax.ShapeDtypeStruct((N, D), jnp.float32),
    )(xch, hrw0, pqp, dis2d, whh1, z, h)


def _heads_body(h_ref, wmu_ref, bmu_ref, wsig_ref, bsig_ref, mu_ref, sig_ref):
    h = h_ref[...]
    mu_ref[...] = jnp.dot(h, wmu_ref[...], preferred_element_type=jnp.float32) + bmu_ref[...]
    s = jnp.dot(h, wsig_ref[...], preferred_element_type=jnp.float32) + bsig_ref[...]
    sig_ref[...] = jnp.logaddexp(s, 0.0)


def _tc_heads(h, wmu, bmu, wsig, bsig):
    return pl.pallas_call(
        _heads_body,
        grid=(NB,),
        in_specs=[
            pl.BlockSpec((BN, D), lambda i: (i, 0)),
            pl.BlockSpec((D, 2), lambda i: (0, 0)),
            pl.BlockSpec((1, 2), lambda i: (0, 0)),
            pl.BlockSpec((D, 2), lambda i: (0, 0)),
            pl.BlockSpec((1, 2), lambda i: (0, 0)),
        ],
        out_specs=[
            pl.BlockSpec((BN, 2), lambda i: (i, 0)),
            pl.BlockSpec((BN, 2), lambda i: (i, 0)),
        ],
        out_shape=[
            jax.ShapeDtypeStruct((N, 2), jnp.float32),
            jax.ShapeDtypeStruct((N, 2), jnp.float32),
        ],
    )(h, wmu, bmu[None, :], wsig, bsig[None, :])


def kernel(in_tensor, edge_index, Wxz, bxz, Whz, bhz, Wxr, bxr, Whr, bhr,
           Wxh, bxh, Whh, bhh, Wmu, bmu, Wsig, bsig):
    src = edge_index[0]
    dst = edge_index[1]
    srcm = _pad_idx(src.reshape(NS, ECW), 0)
    dstm = _prop_indices(dst)
    srcsc = _prop_indices(src)
    x = in_tensor[0]

    zeros128 = jnp.zeros((ARW, D), jnp.float32)
    ones128 = jnp.ones((N, D), jnp.float32)

    deg = _merge_halves(_prop128(ones128, srcm, srcsc, zeros128))[:, 0]
    dis = jnp.where(deg > 0, lax.rsqrt(jnp.maximum(deg, 1e-12)), 0.0)
    dis2d = dis[:, None]

    w0stack = jnp.concatenate([Wxz[0], Wxr[0], Wxh[0]], axis=1)
    w1stack = jnp.concatenate([Wxz[1], Wxr[1], Wxh[1]], axis=1)
    bstack = jnp.concatenate([bxz + bhz, bxr + bhr, bxh + bhh])[None, :]
    wh0zr = jnp.concatenate([Whz[0], Whr[0]], axis=1)
    wh1zr = jnp.concatenate([Whz[1], Whr[1]], axis=1)

    xs, xc = _tc_prep(x, dis2d, w0stack, bstack)
    pxp = jnp.stack([_merge_halves(_prop128(xs[t], srcm, dstm, zeros128))
                     for t in range(T)])
    xcf = _tc_xmerge(pxp, dis2d, xc, w1stack)

    h = _tc_step0(xcf[0])
    for t in range(1, T):
        hs, hw0 = _tc_preh(h, dis2d, wh0zr)
        php = _merge_halves(_prop128(hs, srcm, dstm, zeros128))
        z, hrs, hrw0 = _tc_gates(xcf[t, :, :2 * D], hw0, php, dis2d, wh1zr, h, Whh[0])
        pqp = _merge_halves(_prop128(hrs, srcm, dstm, zeros128))
        h = _tc_newh(xcf[t, :, 2 * D:], hrw0, pqp, dis2d, Whh[1], z, h)

    mu, sig = _tc_heads(h, Wmu, bmu, Wsig, bsig)
    return (mu[None], sig[None])


# speedup vs baseline: 1.5874x; 1.5874x over previous
import functools
import jax
import jax.numpy as jnp
from jax import lax
from jax.experimental import pallas as pl
from jax.experimental.pallas import tpu as pltpu
from jax.experimental.pallas import tpu_sc as plsc

N = 10000
E = 320000
D = 128
T = 4
BN = 2000
NB = N // BN

NC = 2
NS = 16
HN = N // 2
JBASE = HN
JMASK = 1023
AROWS = 6144
ARW = AROWS // NS
ECW = E // NS
C2 = 128
NCHP = 160
IDXROWS = 160
SB = 32
NSB = NCHP // SB


def _make_prop():
    mesh = plsc.VectorSubcoreMesh(core_axis_name="c", subcore_axis_name="s")

    @functools.partial(
        pl.kernel,
        out_type=jax.ShapeDtypeStruct((NC, NS, ARW, D), jnp.float32),
        mesh=mesh,
        scratch_types=[
            pltpu.VMEM((SB, C2), jnp.int32),
            pltpu.VMEM((SB, C2), jnp.int32),
            [pltpu.VMEM((C2, D), jnp.float32)] * 4,
            pltpu.VMEM_SHARED((AROWS, D), jnp.float32),
            [pltpu.SemaphoreType.DMA] * 4,
        ],
    )
    def prop(xs_hbm, srcm_hbm, dstm_hbm, zeros_hbm, out_hbm,
             src_v, dst_v, bufs, acc, gsem):
        c = lax.axis_index("c")
        s = lax.axis_index("s")
        pltpu.sync_copy(zeros_hbm, acc.at[pl.ds(s * ARW, ARW)])
        plsc.subcore_barrier()

        def gath(r, k):
            pltpu.async_copy(xs_hbm.at[src_v.at[r]], bufs[k], gsem[k])

        def wait_g(r, k):
            pltpu.make_async_copy(xs_hbm.at[src_v.at[r]], bufs[k], gsem[k]).wait()

        def scat(r, k):
            pltpu.sync_copy(bufs[k], acc.at[dst_v.at[r]], add=True)

        for m in range(NSB):
            pltpu.sync_copy(srcm_hbm.at[s, pl.ds(m * SB, SB)], src_v)
            pltpu.sync_copy(dstm_hbm.at[c, s, pl.ds(m * SB, SB)], dst_v)
            for k in range(4):
                gath(k, k)

            def body(i2, carry):
                for k in range(4):
                    r = 4 * i2 + k
                    wait_g(r, k)
                    scat(r, k)
                    gath(r + 4, k)
                return carry

            lax.fori_loop(0, SB // 4 - 1, body, 0)
            for k in range(4):
                r = SB - 4 + k
                wait_g(r, k)
                scat(r, k)

        plsc.subcore_barrier()
        pltpu.sync_copy(acc.at[pl.ds(s * ARW, ARW)], out_hbm.at[c, s])

    return prop


_prop128 = _make_prop()


def _pad_idx(a, fill):
    pad = jnp.full((a.shape[0], IDXROWS * C2 - ECW), fill, jnp.int32)
    return jnp.concatenate([a, pad], axis=1).reshape(a.shape[0], IDXROWS, C2)


def _local_dst(d):
    j = JBASE + (d & JMASK)
    l0 = jnp.where(d < HN, d, j)
    l1 = jnp.where(d >= HN, d - HN, j)
    return l0, l1


def _prop_indices(idx):
    l0, l1 = _local_dst(idx)
    return jnp.stack([_pad_idx(l0.reshape(NS, ECW), JBASE),
                      _pad_idx(l1.reshape(NS, ECW), JBASE)])


def _merge_halves(out):
    halves = out.reshape(NC, AROWS, D)
    return jnp.concatenate([halves[0, :HN], halves[1, :HN]], axis=0)


def _prep_body(x_ref, d_ref, w0_ref, b_ref, xs_ref, xc_ref):
    x = x_ref[0]
    d = d_ref[...]
    xs_ref[0] = x * d
    xc_ref[0] = jnp.dot(x, w0_ref[...], preferred_element_type=jnp.float32) + b_ref[...]


def _tc_prep(x, dis2d, w0stack, bstack):
    return pl.pallas_call(
        _prep_body,
        grid=(T, NB),
        in_specs=[
            pl.BlockSpec((1, BN, D), lambda t, i: (t, i, 0)),
            pl.BlockSpec((BN, 1), lambda t, i: (i, 0)),
            pl.BlockSpec((D, 3 * D), lambda t, i: (0, 0)),
            pl.BlockSpec((1, 3 * D), lambda t, i: (0, 0)),
        ],
        out_specs=[
            pl.BlockSpec((1, BN, D), lambda t, i: (t, i, 0)),
            pl.BlockSpec((1, BN, 3 * D), lambda t, i: (t, i, 0)),
        ],
        out_shape=[
            jax.ShapeDtypeStruct((T, N, D), jnp.float32),
            jax.ShapeDtypeStruct((T, N, 3 * D), jnp.float32),
        ],
    )(x, dis2d, w0stack, bstack)


def _xmerge_body(p_ref, d_ref, xc_ref, w1_ref, out_ref):
    p = p_ref[0]
    d = d_ref[...]
    pxs = -(d * p)
    out_ref[0] = xc_ref[0] + jnp.dot(pxs, w1_ref[...], preferred_element_type=jnp.float32)


def _tc_xmerge(pxp, dis2d, xc, w1stack):
    return pl.pallas_call(
        _xmerge_body,
        grid=(T, NB),
        in_specs=[
            pl.BlockSpec((1, BN, D), lambda t, i: (t, i, 0)),
            pl.BlockSpec((BN, 1), lambda t, i: (i, 0)),
            pl.BlockSpec((1, BN, 3 * D), lambda t, i: (t, i, 0)),
            pl.BlockSpec((D, 3 * D), lambda t, i: (0, 0)),
        ],
        out_specs=pl.BlockSpec((1, BN, 3 * D), lambda t, i: (t, i, 0)),
        out_shape=jax.ShapeDtypeStruct((T, N, 3 * D), jnp.float32),
    )(pxp, dis2d, xc, w1stack)


def _step0_body(xc_ref, h_ref):
    c = xc_ref[...]
    z = jax.nn.sigmoid(c[:, :D])
    ht = jnp.tanh(c[:, 2 * D:])
    h_ref[...] = (1.0 - z) * ht


def _tc_step0(xc0):
    return pl.pallas_call(
        _step0_body,
        grid=(NB,),
        in_specs=[pl.BlockSpec((BN, 3 * D), lambda i: (i, 0))],
        out_specs=pl.BlockSpec((BN, D), lambda i: (i, 0)),
        out_shape=jax.ShapeDtypeStruct((N, D), jnp.float32),
    )(xc0)


def _preh_body(h_ref, d_ref, w_ref, hs_ref, hw_ref):
    h = h_ref[...]
    hs_ref[...] = h * d_ref[...]
    hw_ref[...] = jnp.dot(h, w_ref[...], preferred_element_type=jnp.float32)


def _tc_preh(h, dis2d, wh0zr):
    return pl.pallas_call(
        _preh_body,
        grid=(NB,),
        in_specs=[
            pl.BlockSpec((BN, D), lambda i: (i, 0)),
            pl.BlockSpec((BN, 1), lambda i: (i, 0)),
            pl.BlockSpec((D, 2 * D), lambda i: (0, 0)),
        ],
        out_specs=[
            pl.BlockSpec((BN, D), lambda i: (i, 0)),
            pl.BlockSpec((BN, 2 * D), lambda i: (i, 0)),
        ],
        out_shape=[
            jax.ShapeDtypeStruct((N, D), jnp.float32),
            jax.ShapeDtypeStruct((N, 2 * D), jnp.float32),
        ],
    )(h, dis2d, wh0zr)


def _gates_body(xczr_ref, hw_ref, php_ref, d_ref, w1_ref, h_ref, whh0_ref,
                z_ref, hrs_ref, hrw_ref):
    p = php_ref[...]
    d = d_ref[...]
    phs = -(d * p)
    g = xczr_ref[...] + hw_ref[...] + jnp.dot(phs, w1_ref[...],
                                              preferred_element_type=jnp.float32)
    z = jax.nn.sigmoid(g[:, :D])
    r = jax.nn.sigmoid(g[:, D:])
    hr = h_ref[...] * r
    z_ref[...] = z
    hrs_ref[...] = d * hr
    hrw_ref[...] = jnp.dot(hr, whh0_ref[...], preferred_element_type=jnp.float32)


def _tc_gates(xczr, hw0, php, dis2d, wh1zr, h, whh0):
    return pl.pallas_call(
        _gates_body,
        grid=(NB,),
        in_specs=[
            pl.BlockSpec((BN, 2 * D), lambda i: (i, 0)),
            pl.BlockSpec((BN, 2 * D), lambda i: (i, 0)),
            pl.BlockSpec((BN, D), lambda i: (i, 0)),
            pl.BlockSpec((BN, 1), lambda i: (i, 0)),
            pl.BlockSpec((D, 2 * D), lambda i: (0, 0)),
            pl.BlockSpec((BN, D), lambda i: (i, 0)),
            pl.BlockSpec((D, D), lambda i: (0, 0)),
        ],
        out_specs=[
            pl.BlockSpec((BN, D), lambda i: (i, 0)),
            pl.BlockSpec((BN, D), lambda i: (i, 0)),
            pl.BlockSpec((BN, D), lambda i: (i, 0)),
        ],
        out_shape=[
            jax.ShapeDtypeStruct((N, D), jnp.float32),
            jax.ShapeDtypeStruct((N, D), jnp.float32),
            jax.ShapeDtypeStruct((N, D), jnp.float32),
        ],
    )(xczr, hw0, php, dis2d, wh1zr, h, whh0)


def _newh_body(xch_ref, hrw_ref, pqp_ref, d_ref, whh1_ref, z_ref, h_ref, out_ref):
    q = -(d_ref[...] * pqp_ref[...])
    ht = jnp.tanh(xch_ref[...] + hrw_ref[...] +
                  jnp.dot(q, whh1_ref[...], preferred_element_type=jnp.float32))
    z = z_ref[...]
    out_ref[...] = z * h_ref[...] + (1.0 - z) * ht


def _tc_newh(xch, hrw0, pqp, dis2d, whh1, z, h):
    return pl.pallas_call(
        _newh_body,
        grid=(NB,),
        in_specs=[
            pl.BlockSpec((BN, D), lambda i: (i, 0)),
            pl.BlockSpec((BN, D), lambda i: (i, 0)),
            pl.BlockSpec((BN, D), lambda i: (i, 0)),
            pl.BlockSpec((BN, 1), lambda i: (i, 0)),
            pl.BlockSpec((D, D), lambda i: (0, 0)),
            pl.BlockSpec((BN, D), lambda i: (i, 0)),
            pl.BlockSpec((BN, D), lambda i: (i, 0)),
        ],
        out_specs=pl.BlockSpec((BN, D), lambda i: (i, 0)),
        out_shape=jax.ShapeDtypeStruct((N, D), jnp.float32),
    )(xch, hrw0, pqp, dis2d, whh1, z, h)


def _heads_body(h_ref, wmu_ref, bmu_ref, wsig_ref, bsig_ref, mu_ref, sig_ref):
    h = h_ref[...]
    mu_ref[...] = jnp.dot(h, wmu_ref[...], preferred_element_type=jnp.float32) + bmu_ref[...]
    s = jnp.dot(h, wsig_ref[...], preferred_element_type=jnp.float32) + bsig_ref[...]
    sig_ref[...] = jnp.logaddexp(s, 0.0)


def _tc_heads(h, wmu, bmu, wsig, bsig):
    return pl.pallas_call(
        _heads_body,
        grid=(NB,),
        in_specs=[
            pl.BlockSpec((BN, D), lambda i: (i, 0)),
            pl.BlockSpec((D, 2), lambda i: (0, 0)),
            pl.BlockSpec((1, 2), lambda i: (0, 0)),
            pl.BlockSpec((D, 2), lambda i: (0, 0)),
            pl.BlockSpec((1, 2), lambda i: (0, 0)),
        ],
        out_specs=[
            pl.BlockSpec((BN, 2), lambda i: (i, 0)),
            pl.BlockSpec((BN, 2), lambda i: (i, 0)),
        ],
        out_shape=[
            jax.ShapeDtypeStruct((N, 2), jnp.float32),
            jax.ShapeDtypeStruct((N, 2), jnp.float32),
        ],
    )(h, wmu, bmu[None, :], wsig, bsig[None, :])


def kernel(in_tensor, edge_index, Wxz, bxz, Whz, bhz, Wxr, bxr, Whr, bhr,
           Wxh, bxh, Whh, bhh, Wmu, bmu, Wsig, bsig):
    src = edge_index[0]
    dst = edge_index[1]
    srcm = _pad_idx(src.reshape(NS, ECW), 0)
    dstm = _prop_indices(dst)
    srcsc = _prop_indices(src)
    x = in_tensor[0]

    zeros128 = jnp.zeros((ARW, D), jnp.float32)
    ones128 = jnp.ones((N, D), jnp.float32)

    deg = _merge_halves(_prop128(ones128, srcm, srcsc, zeros128))[:, 0]
    dis = jnp.where(deg > 0, lax.rsqrt(jnp.maximum(deg, 1e-12)), 0.0)
    dis2d = dis[:, None]

    w0stack = jnp.concatenate([Wxz[0], Wxr[0], Wxh[0]], axis=1)
    w1stack = jnp.concatenate([Wxz[1], Wxr[1], Wxh[1]], axis=1)
    bstack = jnp.concatenate([bxz + bhz, bxr + bhr, bxh + bhh])[None, :]
    wh0zr = jnp.concatenate([Whz[0], Whr[0]], axis=1)
    wh1zr = jnp.concatenate([Whz[1], Whr[1]], axis=1)

    xs, xc = _tc_prep(x, dis2d, w0stack, bstack)
    pxp = jnp.stack([_merge_halves(_prop128(xs[t], srcm, dstm, zeros128))
                     for t in range(T)])
    xcf = _tc_xmerge(pxp, dis2d, xc, w1stack)

    h = _tc_step0(xcf[0])
    for t in range(1, T):
        hs, hw0 = _tc_preh(h, dis2d, wh0zr)
        php = _merge_halves(_prop128(hs, srcm, dstm, zeros128))
        z, hrs, hrw0 = _tc_gates(xcf[t, :, :2 * D], hw0, php, dis2d, wh1zr, h, Whh[0])
        pqp = _merge_halves(_prop128(hrs, srcm, dstm, zeros128))
        h = _tc_newh(xcf[t, :, 2 * D:], hrw0, pqp, dis2d, Whh[1], z, h)

    mu, sig = _tc_heads(h, Wmu, bmu, Wsig, bsig)
    return (mu[None], sig[None])


# speedup vs baseline: 5.6100x; 3.5341x over previous
import functools
import jax
import jax.numpy as jnp
from jax import lax
from jax.experimental import pallas as pl
from jax.experimental.pallas import tpu as pltpu
from jax.experimental.pallas import tpu_sc as plsc

N = 10000
E = 320000
D = 128
T = 4
BN = 2000
NB = N // BN

NC = 2
NS = 16
HN = N // 2
JBASE = HN
JMASK = 1023
AROWS = 6144
ARW = AROWS // NS
ECW = E // NS
C2 = 128
NCHP = 160
IDXROWS = 160
SB = 32
NSB = NCHP // SB


def _make_prop():
    mesh = plsc.VectorSubcoreMesh(core_axis_name="c", subcore_axis_name="s")

    @functools.partial(
        pl.kernel,
        out_type=jax.ShapeDtypeStruct((NC, NS, ARW, D), jnp.float32),
        mesh=mesh,
        scratch_types=[
            pltpu.VMEM((SB, C2), jnp.int32),
            pltpu.VMEM((SB, C2), jnp.int32),
            [pltpu.VMEM((C2, D), jnp.float32)] * 4,
            pltpu.VMEM_SHARED((AROWS, D), jnp.float32),
            [pltpu.SemaphoreType.DMA] * 4,
        ],
    )
    def prop(xs_hbm, srcm_hbm, dstm_hbm, zeros_hbm, out_hbm,
             src_v, dst_v, bufs, acc, gsem):
        c = lax.axis_index("c")
        s = lax.axis_index("s")
        pltpu.sync_copy(zeros_hbm, acc.at[pl.ds(s * ARW, ARW)])
        plsc.subcore_barrier()

        def gath(r, k):
            pltpu.async_copy(xs_hbm.at[src_v.at[r]], bufs[k], gsem[k])

        def wait_g(r, k):
            pltpu.make_async_copy(xs_hbm.at[src_v.at[r]], bufs[k], gsem[k]).wait()

        def scat(r, k):
            pltpu.sync_copy(bufs[k], acc.at[dst_v.at[r]], add=True)

        for m in range(NSB):
            pltpu.sync_copy(srcm_hbm.at[s, pl.ds(m * SB, SB)], src_v)
            pltpu.sync_copy(dstm_hbm.at[c, s, pl.ds(m * SB, SB)], dst_v)
            for k in range(4):
                gath(k, k)

            def body(i2, carry):
                for k in range(4):
                    r = 4 * i2 + k
                    wait_g(r, k)
                    scat(r, k)
                    gath(r + 4, k)
                return carry

            lax.fori_loop(0, SB // 4 - 1, body, 0)
            for k in range(4):
                r = SB - 4 + k
                wait_g(r, k)
                scat(r, k)

        plsc.subcore_barrier()
        pltpu.sync_copy(acc.at[pl.ds(s * ARW, ARW)], out_hbm.at[c, s])

    return prop


_prop128 = _make_prop()


def _pad_idx(a, fill):
    npad = IDXROWS * C2 - ECW
    if fill is None:
        pad = jnp.broadcast_to(jnp.arange(npad, dtype=jnp.int32) % N,
                               (a.shape[0], npad))
    elif fill == -1:
        pad = jnp.broadcast_to(
            JBASE + (jnp.arange(npad, dtype=jnp.int32) & JMASK),
            (a.shape[0], npad))
    else:
        pad = jnp.full((a.shape[0], npad), fill, jnp.int32)
    return jnp.concatenate([a, pad], axis=1).reshape(a.shape[0], IDXROWS, C2)


def _local_dst(d):
    j = JBASE + (d & JMASK)
    l0 = jnp.where(d < HN, d, j)
    l1 = jnp.where(d >= HN, d - HN, j)
    return l0, l1


def _prop_indices(idx):
    l0, l1 = _local_dst(idx)
    return jnp.stack([_pad_idx(l0.reshape(NS, ECW), -1),
                      _pad_idx(l1.reshape(NS, ECW), -1)])


def _merge_halves(out):
    halves = out.reshape(NC, AROWS, D)
    return jnp.concatenate([halves[0, :HN], halves[1, :HN]], axis=0)


def _prep_body(x_ref, d_ref, w0_ref, b_ref, xs_ref, xc_ref):
    x = x_ref[0]
    d = d_ref[...]
    xs_ref[0] = x * d
    xc_ref[0] = jnp.dot(x, w0_ref[...], preferred_element_type=jnp.float32) + b_ref[...]


def _tc_prep(x, dis2d, w0stack, bstack):
    return pl.pallas_call(
        _prep_body,
        grid=(T, NB),
        in_specs=[
            pl.BlockSpec((1, BN, D), lambda t, i: (t, i, 0)),
            pl.BlockSpec((BN, 1), lambda t, i: (i, 0)),
            pl.BlockSpec((D, 3 * D), lambda t, i: (0, 0)),
            pl.BlockSpec((1, 3 * D), lambda t, i: (0, 0)),
        ],
        out_specs=[
            pl.BlockSpec((1, BN, D), lambda t, i: (t, i, 0)),
            pl.BlockSpec((1, BN, 3 * D), lambda t, i: (t, i, 0)),
        ],
        out_shape=[
            jax.ShapeDtypeStruct((T, N, D), jnp.float32),
            jax.ShapeDtypeStruct((T, N, 3 * D), jnp.float32),
        ],
    )(x, dis2d, w0stack, bstack)


def _xmerge_body(p_ref, d_ref, xc_ref, w1_ref, out_ref):
    p = p_ref[0]
    d = d_ref[...]
    pxs = -(d * p)
    out_ref[0] = xc_ref[0] + jnp.dot(pxs, w1_ref[...], preferred_element_type=jnp.float32)


def _tc_xmerge(pxp, dis2d, xc, w1stack):
    return pl.pallas_call(
        _xmerge_body,
        grid=(T, NB),
        in_specs=[
            pl.BlockSpec((1, BN, D), lambda t, i: (t, i, 0)),
            pl.BlockSpec((BN, 1), lambda t, i: (i, 0)),
            pl.BlockSpec((1, BN, 3 * D), lambda t, i: (t, i, 0)),
            pl.BlockSpec((D, 3 * D), lambda t, i: (0, 0)),
        ],
        out_specs=pl.BlockSpec((1, BN, 3 * D), lambda t, i: (t, i, 0)),
        out_shape=jax.ShapeDtypeStruct((T, N, 3 * D), jnp.float32),
    )(pxp, dis2d, xc, w1stack)


def _step0_body(xc_ref, h_ref):
    c = xc_ref[...]
    z = jax.nn.sigmoid(c[:, :D])
    ht = jnp.tanh(c[:, 2 * D:])
    h_ref[...] = (1.0 - z) * ht


def _tc_step0(xc0):
    return pl.pallas_call(
        _step0_body,
        grid=(NB,),
        in_specs=[pl.BlockSpec((BN, 3 * D), lambda i: (i, 0))],
        out_specs=pl.BlockSpec((BN, D), lambda i: (i, 0)),
        out_shape=jax.ShapeDtypeStruct((N, D), jnp.float32),
    )(xc0)


def _preh_body(h_ref, d_ref, w_ref, hs_ref, hw_ref):
    h = h_ref[...]
    hs_ref[...] = h * d_ref[...]
    hw_ref[...] = jnp.dot(h, w_ref[...], preferred_element_type=jnp.float32)


def _tc_preh(h, dis2d, wh0zr):
    return pl.pallas_call(
        _preh_body,
        grid=(NB,),
        in_specs=[
            pl.BlockSpec((BN, D), lambda i: (i, 0)),
            pl.BlockSpec((BN, 1), lambda i: (i, 0)),
            pl.BlockSpec((D, 2 * D), lambda i: (0, 0)),
        ],
        out_specs=[
            pl.BlockSpec((BN, D), lambda i: (i, 0)),
            pl.BlockSpec((BN, 2 * D), lambda i: (i, 0)),
        ],
        out_shape=[
            jax.ShapeDtypeStruct((N, D), jnp.float32),
            jax.ShapeDtypeStruct((N, 2 * D), jnp.float32),
        ],
    )(h, dis2d, wh0zr)


def _gates_body(xczr_ref, hw_ref, php_ref, d_ref, w1_ref, h_ref, whh0_ref,
                z_ref, hrs_ref, hrw_ref):
    p = php_ref[...]
    d = d_ref[...]
    phs = -(d * p)
    g = xczr_ref[...] + hw_ref[...] + jnp.dot(phs, w1_ref[...],
                                              preferred_element_type=jnp.float32)
    z = jax.nn.sigmoid(g[:, :D])
    r = jax.nn.sigmoid(g[:, D:])
    hr = h_ref[...] * r
    z_ref[...] = z
    hrs_ref[...] = d * hr
    hrw_ref[...] = jnp.dot(hr, whh0_ref[...], preferred_element_type=jnp.float32)


def _tc_gates(xczr, hw0, php, dis2d, wh1zr, h, whh0):
    return pl.pallas_call(
        _gates_body,
        grid=(NB,),
        in_specs=[
            pl.BlockSpec((BN, 2 * D), lambda i: (i, 0)),
            pl.BlockSpec((BN, 2 * D), lambda i: (i, 0)),
            pl.BlockSpec((BN, D), lambda i: (i, 0)),
            pl.BlockSpec((BN, 1), lambda i: (i, 0)),
            pl.BlockSpec((D, 2 * D), lambda i: (0, 0)),
            pl.BlockSpec((BN, D), lambda i: (i, 0)),
            pl.BlockSpec((D, D), lambda i: (0, 0)),
        ],
        out_specs=[
            pl.BlockSpec((BN, D), lambda i: (i, 0)),
            pl.BlockSpec((BN, D), lambda i: (i, 0)),
            pl.BlockSpec((BN, D), lambda i: (i, 0)),
        ],
        out_shape=[
            jax.ShapeDtypeStruct((N, D), jnp.float32),
            jax.ShapeDtypeStruct((N, D), jnp.float32),
            jax.ShapeDtypeStruct((N, D), jnp.float32),
        ],
    )(xczr, hw0, php, dis2d, wh1zr, h, whh0)


def _newh_body(xch_ref, hrw_ref, pqp_ref, d_ref, whh1_ref, z_ref, h_ref, out_ref):
    q = -(d_ref[...] * pqp_ref[...])
    ht = jnp.tanh(xch_ref[...] + hrw_ref[...] +
                  jnp.dot(q, whh1_ref[...], preferred_element_type=jnp.float32))
    z = z_ref[...]
    out_ref[...] = z * h_ref[...] + (1.0 - z) * ht


def _tc_newh(xch, hrw0, pqp, dis2d, whh1, z, h):
    return pl.pallas_call(
        _newh_body,
        grid=(NB,),
        in_specs=[
            pl.BlockSpec((BN, D), lambda i: (i, 0)),
            pl.BlockSpec((BN, D), lambda i: (i, 0)),
            pl.BlockSpec((BN, D), lambda i: (i, 0)),
            pl.BlockSpec((BN, 1), lambda i: (i, 0)),
            pl.BlockSpec((D, D), lambda i: (0, 0)),
            pl.BlockSpec((BN, D), lambda i: (i, 0)),
            pl.BlockSpec((BN, D), lambda i: (i, 0)),
        ],
        out_specs=pl.BlockSpec((BN, D), lambda i: (i, 0)),
        out_shape=jax.ShapeDtypeStruct((N, D), jnp.float32),
    )(xch, hrw0, pqp, dis2d, whh1, z, h)


def _heads_body(h_ref, wmu_ref, bmu_ref, wsig_ref, bsig_ref, mu_ref, sig_ref):
    h = h_ref[...]
    mu_ref[...] = jnp.dot(h, wmu_ref[...], preferred_element_type=jnp.float32) + bmu_ref[...]
    s = jnp.dot(h, wsig_ref[...], preferred_element_type=jnp.float32) + bsig_ref[...]
    sig_ref[...] = jnp.logaddexp(s, 0.0)


def _tc_heads(h, wmu, bmu, wsig, bsig):
    return pl.pallas_call(
        _heads_body,
        grid=(NB,),
        in_specs=[
            pl.BlockSpec((BN, D), lambda i: (i, 0)),
            pl.BlockSpec((D, 2), lambda i: (0, 0)),
            pl.BlockSpec((1, 2), lambda i: (0, 0)),
            pl.BlockSpec((D, 2), lambda i: (0, 0)),
            pl.BlockSpec((1, 2), lambda i: (0, 0)),
        ],
        out_specs=[
            pl.BlockSpec((BN, 2), lambda i: (i, 0)),
            pl.BlockSpec((BN, 2), lambda i: (i, 0)),
        ],
        out_shape=[
            jax.ShapeDtypeStruct((N, 2), jnp.float32),
            jax.ShapeDtypeStruct((N, 2), jnp.float32),
        ],
    )(h, wmu, bmu[None, :], wsig, bsig[None, :])


def kernel(in_tensor, edge_index, Wxz, bxz, Whz, bhz, Wxr, bxr, Whr, bhr,
           Wxh, bxh, Whh, bhh, Wmu, bmu, Wsig, bsig):
    src = edge_index[0]
    dst = edge_index[1]
    srcm = _pad_idx(src.reshape(NS, ECW), None)
    dstm = _prop_indices(dst)
    srcsc = _prop_indices(src)
    x = in_tensor[0]

    zeros128 = jnp.zeros((ARW, D), jnp.float32)
    ones128 = jnp.ones((N, D), jnp.float32)

    deg = _merge_halves(_prop128(ones128, srcm, srcsc, zeros128))[:, 0]
    dis = jnp.where(deg > 0, lax.rsqrt(jnp.maximum(deg, 1e-12)), 0.0)
    dis2d = dis[:, None]

    w0stack = jnp.concatenate([Wxz[0], Wxr[0], Wxh[0]], axis=1)
    w1stack = jnp.concatenate([Wxz[1], Wxr[1], Wxh[1]], axis=1)
    bstack = jnp.concatenate([bxz + bhz, bxr + bhr, bxh + bhh])[None, :]
    wh0zr = jnp.concatenate([Whz[0], Whr[0]], axis=1)
    wh1zr = jnp.concatenate([Whz[1], Whr[1]], axis=1)

    xs, xc = _tc_prep(x, dis2d, w0stack, bstack)
    pxp = jnp.stack([_merge_halves(_prop128(xs[t], srcm, dstm, zeros128))
                     for t in range(T)])
    xcf = _tc_xmerge(pxp, dis2d, xc, w1stack)

    h = _tc_step0(xcf[0])
    for t in range(1, T):
        hs, hw0 = _tc_preh(h, dis2d, wh0zr)
        php = _merge_halves(_prop128(hs, srcm, dstm, zeros128))
        z, hrs, hrw0 = _tc_gates(xcf[t, :, :2 * D], hw0, php, dis2d, wh1zr, h, Whh[0])
        pqp = _merge_halves(_prop128(hrs, srcm, dstm, zeros128))
        h = _tc_newh(xcf[t, :, 2 * D:], hrw0, pqp, dis2d, Whh[1], z, h)

    mu, sig = _tc_heads(h, Wmu, bmu, Wsig, bsig)
    return (mu[None], sig[None])


# speedup vs baseline: 5.6485x; 1.0069x over previous
import functools
import jax
import jax.numpy as jnp
from jax import lax
from jax.experimental import pallas as pl
from jax.experimental.pallas import tpu as pltpu
from jax.experimental.pallas import tpu_sc as plsc

N = 10000
E = 320000
D = 128
T = 4
BN = 2000
NB = N // BN

NC = 2
NS = 16
HN = N // 2
JBASE = HN
JMASK = 1023
AROWS = 6144
ARW = AROWS // NS
ECW = E // NS
C2 = 128
NCHP = 160
IDXROWS = 160
SB = 32
NSB = NCHP // SB


def _make_prop():
    mesh = plsc.VectorSubcoreMesh(core_axis_name="c", subcore_axis_name="s")

    @functools.partial(
        pl.kernel,
        out_type=jax.ShapeDtypeStruct((NC, NS, ARW, D), jnp.float32),
        mesh=mesh,
        scratch_types=[
            pltpu.VMEM((SB, C2), jnp.int32),
            pltpu.VMEM((SB, C2), jnp.int32),
            [pltpu.VMEM((C2, D), jnp.float32)] * 4,
            pltpu.VMEM_SHARED((AROWS, D), jnp.float32),
            [pltpu.SemaphoreType.DMA] * 4,
        ],
    )
    def prop(xs_hbm, srcm_hbm, dstm_hbm, zeros_hbm, out_hbm,
             src_v, dst_v, bufs, acc, gsem):
        c = lax.axis_index("c")
        s = lax.axis_index("s")
        pltpu.sync_copy(zeros_hbm, acc.at[pl.ds(s * ARW, ARW)])
        plsc.subcore_barrier()

        def gath(r, k):
            pltpu.async_copy(xs_hbm.at[src_v.at[r]], bufs[k], gsem[k])

        def wait_g(r, k):
            pltpu.make_async_copy(xs_hbm.at[src_v.at[r]], bufs[k], gsem[k]).wait()

        def scat(r, k):
            pltpu.sync_copy(bufs[k], acc.at[dst_v.at[r]], add=True)

        for m in range(NSB):
            pltpu.sync_copy(srcm_hbm.at[s, pl.ds(m * SB, SB)], src_v)
            pltpu.sync_copy(dstm_hbm.at[c, s, pl.ds(m * SB, SB)], dst_v)
            for k in range(4):
                gath(k, k)

            def body(i2, carry):
                for k in range(4):
                    r = 4 * i2 + k
                    wait_g(r, k)
                    scat(r, k)
                    gath(r + 4, k)
                return carry

            lax.fori_loop(0, SB // 4 - 1, body, 0)
            for k in range(4):
                r = SB - 4 + k
                wait_g(r, k)
                scat(r, k)

        plsc.subcore_barrier()
        pltpu.sync_copy(acc.at[pl.ds(s * ARW, ARW)], out_hbm.at[c, s])

    return prop


_prop128 = _make_prop()


def _make_prop4():
    mesh = plsc.VectorSubcoreMesh(core_axis_name="c", subcore_axis_name="s")
    out1 = jax.ShapeDtypeStruct((NC, NS, ARW, D), jnp.float32)

    @functools.partial(
        pl.kernel,
        out_type=[out1] * 4,
        mesh=mesh,
        scratch_types=[
            pltpu.VMEM((SB, C2), jnp.int32),
            pltpu.VMEM((SB, C2), jnp.int32),
            [pltpu.VMEM((C2, D), jnp.float32)] * 4,
            pltpu.VMEM_SHARED((AROWS, D), jnp.float32),
            [pltpu.SemaphoreType.DMA] * 4,
        ],
    )
    def prop4(t1, t2, t3, t4, srcm_hbm, dstm_hbm, zeros_hbm,
              o1, o2, o3, o4, src_v, dst_v, bufs, acc, gsem):
        c = lax.axis_index("c")
        s = lax.axis_index("s")

        def one(tab, scat_hbm, out_hbm):
            pltpu.sync_copy(zeros_hbm, acc.at[pl.ds(s * ARW, ARW)])
            plsc.subcore_barrier()

            def gath(r, k):
                pltpu.async_copy(tab.at[src_v.at[r]], bufs[k], gsem[k])

            def wait_g(r, k):
                pltpu.make_async_copy(tab.at[src_v.at[r]], bufs[k],
                                      gsem[k]).wait()

            def scat(r, k):
                pltpu.sync_copy(bufs[k], acc.at[dst_v.at[r]], add=True)

            for m in range(NSB):
                pltpu.sync_copy(srcm_hbm.at[s, pl.ds(m * SB, SB)], src_v)
                pltpu.sync_copy(scat_hbm.at[c, s, pl.ds(m * SB, SB)], dst_v)
                for k in range(4):
                    gath(k, k)

                def body(i2, carry):
                    for k in range(4):
                        r = 4 * i2 + k
                        wait_g(r, k)
                        scat(r, k)
                        gath(r + 4, k)
                    return carry

                lax.fori_loop(0, SB // 4 - 1, body, 0)
                for k in range(4):
                    r = SB - 4 + k
                    wait_g(r, k)
                    scat(r, k)

            plsc.subcore_barrier()
            pltpu.sync_copy(acc.at[pl.ds(s * ARW, ARW)], out_hbm.at[c, s])
            plsc.subcore_barrier()

        one(t1, dstm_hbm, o1)
        one(t2, dstm_hbm, o2)
        one(t3, dstm_hbm, o3)
        one(t4, dstm_hbm, o4)

    return prop4


_prop4 = _make_prop4()


def _pad_idx(a, fill):
    npad = IDXROWS * C2 - ECW
    if fill is None:
        pad = jnp.broadcast_to(jnp.arange(npad, dtype=jnp.int32) % N,
                               (a.shape[0], npad))
    elif fill == -1:
        pad = jnp.broadcast_to(
            JBASE + (jnp.arange(npad, dtype=jnp.int32) & JMASK),
            (a.shape[0], npad))
    else:
        pad = jnp.full((a.shape[0], npad), fill, jnp.int32)
    return jnp.concatenate([a, pad], axis=1).reshape(a.shape[0], IDXROWS, C2)


def _local_dst(d):
    j = JBASE + (d & JMASK)
    l0 = jnp.where(d < HN, d, j)
    l1 = jnp.where(d >= HN, d - HN, j)
    return l0, l1


def _prop_indices(idx):
    l0, l1 = _local_dst(idx)
    return jnp.stack([_pad_idx(l0.reshape(NS, ECW), -1),
                      _pad_idx(l1.reshape(NS, ECW), -1)])


def _merge_halves(out):
    halves = out.reshape(NC, AROWS, D)
    return jnp.concatenate([halves[0, :HN], halves[1, :HN]], axis=0)


def _prep_body(x_ref, d_ref, w0_ref, b_ref, xs_ref, xc_ref):
    x = x_ref[0]
    d = d_ref[...]
    xs_ref[0] = x * d
    xc_ref[0] = jnp.dot(x, w0_ref[...], preferred_element_type=jnp.float32) + b_ref[...]


def _tc_prep(x, dis2d, w0stack, bstack):
    return pl.pallas_call(
        _prep_body,
        grid=(T, NB),
        in_specs=[
            pl.BlockSpec((1, BN, D), lambda t, i: (t, i, 0)),
            pl.BlockSpec((BN, 1), lambda t, i: (i, 0)),
            pl.BlockSpec((D, 3 * D), lambda t, i: (0, 0)),
            pl.BlockSpec((1, 3 * D), lambda t, i: (0, 0)),
        ],
        out_specs=[
            pl.BlockSpec((1, BN, D), lambda t, i: (t, i, 0)),
            pl.BlockSpec((1, BN, 3 * D), lambda t, i: (t, i, 0)),
        ],
        out_shape=[
            jax.ShapeDtypeStruct((T, N, D), jnp.float32),
            jax.ShapeDtypeStruct((T, N, 3 * D), jnp.float32),
        ],
    )(x, dis2d, w0stack, bstack)


def _xmerge_body(p_ref, d_ref, xc_ref, w1_ref, out_ref):
    p = p_ref[0]
    d = d_ref[...]
    pxs = -(d * p)
    out_ref[0] = xc_ref[0] + jnp.dot(pxs, w1_ref[...], preferred_element_type=jnp.float32)


def _tc_xmerge(pxp, dis2d, xc, w1stack):
    return pl.pallas_call(
        _xmerge_body,
        grid=(T, NB),
        in_specs=[
            pl.BlockSpec((1, BN, D), lambda t, i: (t, i, 0)),
            pl.BlockSpec((BN, 1), lambda t, i: (i, 0)),
            pl.BlockSpec((1, BN, 3 * D), lambda t, i: (t, i, 0)),
            pl.BlockSpec((D, 3 * D), lambda t, i: (0, 0)),
        ],
        out_specs=pl.BlockSpec((1, BN, 3 * D), lambda t, i: (t, i, 0)),
        out_shape=jax.ShapeDtypeStruct((T, N, 3 * D), jnp.float32),
    )(pxp, dis2d, xc, w1stack)


def _step0_body(xc_ref, h_ref):
    c = xc_ref[...]
    z = jax.nn.sigmoid(c[:, :D])
    ht = jnp.tanh(c[:, 2 * D:])
    h_ref[...] = (1.0 - z) * ht


def _tc_step0(xc0):
    return pl.pallas_call(
        _step0_body,
        grid=(NB,),
        in_specs=[pl.BlockSpec((BN, 3 * D), lambda i: (i, 0))],
        out_specs=pl.BlockSpec((BN, D), lambda i: (i, 0)),
        out_shape=jax.ShapeDtypeStruct((N, D), jnp.float32),
    )(xc0)


def _preh_body(h_ref, d_ref, w_ref, hs_ref, hw_ref):
    h = h_ref[...]
    hs_ref[...] = h * d_ref[...]
    hw_ref[...] = jnp.dot(h, w_ref[...], preferred_element_type=jnp.float32)


def _tc_preh(h, dis2d, wh0zr):
    return pl.pallas_call(
        _preh_body,
        grid=(NB,),
        in_specs=[
            pl.BlockSpec((BN, D), lambda i: (i, 0)),
            pl.BlockSpec((BN, 1), lambda i: (i, 0)),
            pl.BlockSpec((D, 2 * D), lambda i: (0, 0)),
        ],
        out_specs=[
            pl.BlockSpec((BN, D), lambda i: (i, 0)),
            pl.BlockSpec((BN, 2 * D), lambda i: (i, 0)),
        ],
        out_shape=[
            jax.ShapeDtypeStruct((N, D), jnp.float32),
            jax.ShapeDtypeStruct((N, 2 * D), jnp.float32),
        ],
    )(h, dis2d, wh0zr)


def _gates_body(xczr_ref, hw_ref, php_ref, d_ref, w1_ref, h_ref, whh0_ref,
                z_ref, hrs_ref, hrw_ref):
    p = php_ref[...]
    d = d_ref[...]
    phs = -(d * p)
    g = xczr_ref[...] + hw_ref[...] + jnp.dot(phs, w1_ref[...],
                                              preferred_element_type=jnp.float32)
    z = jax.nn.sigmoid(g[:, :D])
    r = jax.nn.sigmoid(g[:, D:])
    hr = h_ref[...] * r
    z_ref[...] = z
    hrs_ref[...] = d * hr
    hrw_ref[...] = jnp.dot(hr, whh0_ref[...], preferred_element_type=jnp.float32)


def _tc_gates(xczr, hw0, php, dis2d, wh1zr, h, whh0):
    return pl.pallas_call(
        _gates_body,
        grid=(NB,),
        in_specs=[
            pl.BlockSpec((BN, 2 * D), lambda i: (i, 0)),
            pl.BlockSpec((BN, 2 * D), lambda i: (i, 0)),
            pl.BlockSpec((BN, D), lambda i: (i, 0)),
            pl.BlockSpec((BN, 1), lambda i: (i, 0)),
            pl.BlockSpec((D, 2 * D), lambda i: (0, 0)),
            pl.BlockSpec((BN, D), lambda i: (i, 0)),
            pl.BlockSpec((D, D), lambda i: (0, 0)),
        ],
        out_specs=[
            pl.BlockSpec((BN, D), lambda i: (i, 0)),
            pl.BlockSpec((BN, D), lambda i: (i, 0)),
            pl.BlockSpec((BN, D), lambda i: (i, 0)),
        ],
        out_shape=[
            jax.ShapeDtypeStruct((N, D), jnp.float32),
            jax.ShapeDtypeStruct((N, D), jnp.float32),
            jax.ShapeDtypeStruct((N, D), jnp.float32),
        ],
    )(xczr, hw0, php, dis2d, wh1zr, h, whh0)


def _newh_body(xch_ref, hrw_ref, pqp_ref, d_ref, whh1_ref, z_ref, h_ref, out_ref):
    q = -(d_ref[...] * pqp_ref[...])
    ht = jnp.tanh(xch_ref[...] + hrw_ref[...] +
                  jnp.dot(q, whh1_ref[...], preferred_element_type=jnp.float32))
    z = z_ref[...]
    out_ref[...] = z * h_ref[...] + (1.0 - z) * ht


def _tc_newh(xch, hrw0, pqp, dis2d, whh1, z, h):
    return pl.pallas_call(
        _newh_body,
        grid=(NB,),
        in_specs=[
            pl.BlockSpec((BN, D), lambda i: (i, 0)),
            pl.BlockSpec((BN, D), lambda i: (i, 0)),
            pl.BlockSpec((BN, D), lambda i: (i, 0)),
            pl.BlockSpec((BN, 1), lambda i: (i, 0)),
            pl.BlockSpec((D, D), lambda i: (0, 0)),
            pl.BlockSpec((BN, D), lambda i: (i, 0)),
            pl.BlockSpec((BN, D), lambda i: (i, 0)),
        ],
        out_specs=pl.BlockSpec((BN, D), lambda i: (i, 0)),
        out_shape=jax.ShapeDtypeStruct((N, D), jnp.float32),
    )(xch, hrw0, pqp, dis2d, whh1, z, h)


def _heads_body(h_ref, wmu_ref, bmu_ref, wsig_ref, bsig_ref, mu_ref, sig_ref):
    h = h_ref[...]
    mu_ref[...] = jnp.dot(h, wmu_ref[...], preferred_element_type=jnp.float32) + bmu_ref[...]
    s = jnp.dot(h, wsig_ref[...], preferred_element_type=jnp.float32) + bsig_ref[...]
    sig_ref[...] = jnp.logaddexp(s, 0.0)


def _tc_heads(h, wmu, bmu, wsig, bsig):
    return pl.pallas_call(
        _heads_body,
        grid=(NB,),
        in_specs=[
            pl.BlockSpec((BN, D), lambda i: (i, 0)),
            pl.BlockSpec((D, 2), lambda i: (0, 0)),
            pl.BlockSpec((1, 2), lambda i: (0, 0)),
            pl.BlockSpec((D, 2), lambda i: (0, 0)),
            pl.BlockSpec((1, 2), lambda i: (0, 0)),
        ],
        out_specs=[
            pl.BlockSpec((BN, 2), lambda i: (i, 0)),
            pl.BlockSpec((BN, 2), lambda i: (i, 0)),
        ],
        out_shape=[
            jax.ShapeDtypeStruct((N, 2), jnp.float32),
            jax.ShapeDtypeStruct((N, 2), jnp.float32),
        ],
    )(h, wmu, bmu[None, :], wsig, bsig[None, :])


def kernel(in_tensor, edge_index, Wxz, bxz, Whz, bhz, Wxr, bxr, Whr, bhr,
           Wxh, bxh, Whh, bhh, Wmu, bmu, Wsig, bsig):
    src = edge_index[0]
    dst = edge_index[1]
    srcm = _pad_idx(src.reshape(NS, ECW), None)
    dstm = _prop_indices(dst)
    srcsc = _prop_indices(src)
    x = in_tensor[0]

    zeros128 = jnp.zeros((ARW, D), jnp.float32)
    ones128 = jnp.ones((N, D), jnp.float32)

    deg = _merge_halves(_prop128(ones128, srcm, srcsc, zeros128))[:, 0]
    dis = jnp.where(deg > 0, lax.rsqrt(jnp.maximum(deg, 1e-12)), 0.0)
    dis2d = dis[:, None]

    w0stack = jnp.concatenate([Wxz[0], Wxr[0], Wxh[0]], axis=1)
    w1stack = jnp.concatenate([Wxz[1], Wxr[1], Wxh[1]], axis=1)
    bstack = jnp.concatenate([bxz + bhz, bxr + bhr, bxh + bhh])[None, :]
    wh0zr = jnp.concatenate([Whz[0], Whr[0]], axis=1)
    wh1zr = jnp.concatenate([Whz[1], Whr[1]], axis=1)

    xs, xc = _tc_prep(x, dis2d, w0stack, bstack)
    pxo = _prop4(xs[0], xs[1], xs[2], xs[3], srcm, dstm, zeros128)
    pxp = jnp.stack([_merge_halves(o) for o in pxo])
    xcf = _tc_xmerge(pxp, dis2d, xc, w1stack)

    h = _tc_step0(xcf[0])
    for t in range(1, T):
        hs, hw0 = _tc_preh(h, dis2d, wh0zr)
        php = _merge_halves(_prop128(hs, srcm, dstm, zeros128))
        z, hrs, hrw0 = _tc_gates(xcf[t, :, :2 * D], hw0, php, dis2d, wh1zr, h, Whh[0])
        pqp = _merge_halves(_prop128(hrs, srcm, dstm, zeros128))
        h = _tc_newh(xcf[t, :, 2 * D:], hrw0, pqp, dis2d, Whh[1], z, h)

    mu, sig = _tc_heads(h, Wmu, bmu, Wsig, bsig)
    return (mu[None], sig[None])


# speedup vs baseline: 5.7601x; 1.0198x over previous
import functools
import jax
import jax.numpy as jnp
from jax import lax
from jax.experimental import pallas as pl
from jax.experimental.pallas import tpu as pltpu
from jax.experimental.pallas import tpu_sc as plsc

N = 10000
E = 320000
D = 128
T = 4
BN = 2000
NB = N // BN

NC = 2
NS = 16
HN = N // 2
JBASE = HN
JMASK = 1023
AROWS = 6144
ARW = AROWS // NS
ECW = E // NS
C2 = 128
NCHP = 160
IDXROWS = 160
SB = 40
NSB = NCHP // SB


def _make_prop():
    mesh = plsc.VectorSubcoreMesh(core_axis_name="c", subcore_axis_name="s")

    @functools.partial(
        pl.kernel,
        out_type=jax.ShapeDtypeStruct((NC, NS, ARW, D), jnp.float32),
        mesh=mesh,
        scratch_types=[
            pltpu.VMEM((SB, C2), jnp.int32),
            pltpu.VMEM((SB, C2), jnp.int32),
            [pltpu.VMEM((C2, D), jnp.float32)] * 4,
            pltpu.VMEM_SHARED((AROWS, D), jnp.float32),
            [pltpu.SemaphoreType.DMA] * 4,
        ],
    )
    def prop(xs_hbm, srcm_hbm, dstm_hbm, zeros_hbm, out_hbm,
             src_v, dst_v, bufs, acc, gsem):
        c = lax.axis_index("c")
        s = lax.axis_index("s")
        pltpu.sync_copy(zeros_hbm, acc.at[pl.ds(s * ARW, ARW)])
        plsc.subcore_barrier()

        def gath(r, k):
            pltpu.async_copy(xs_hbm.at[src_v.at[r]], bufs[k], gsem[k])

        def wait_g(r, k):
            pltpu.make_async_copy(xs_hbm.at[src_v.at[r]], bufs[k], gsem[k]).wait()

        def scat(r, k):
            pltpu.sync_copy(bufs[k], acc.at[dst_v.at[r]], add=True)

        for m in range(NSB):
            pltpu.sync_copy(srcm_hbm.at[s, pl.ds(m * SB, SB)], src_v)
            pltpu.sync_copy(dstm_hbm.at[c, s, pl.ds(m * SB, SB)], dst_v)
            for k in range(4):
                gath(k, k)

            def body(i2, carry):
                for k in range(4):
                    r = 4 * i2 + k
                    wait_g(r, k)
                    scat(r, k)
                    gath(r + 4, k)
                return carry

            lax.fori_loop(0, SB // 4 - 1, body, 0)
            for k in range(4):
                r = SB - 4 + k
                wait_g(r, k)
                scat(r, k)

        plsc.subcore_barrier()
        pltpu.sync_copy(acc.at[pl.ds(s * ARW, ARW)], out_hbm.at[c, s])

    return prop


_prop128 = _make_prop()


def _make_prop4():
    mesh = plsc.VectorSubcoreMesh(core_axis_name="c", subcore_axis_name="s")
    out1 = jax.ShapeDtypeStruct((NC, NS, ARW, D), jnp.float32)

    @functools.partial(
        pl.kernel,
        out_type=[out1] * 4,
        mesh=mesh,
        scratch_types=[
            pltpu.VMEM((SB, C2), jnp.int32),
            pltpu.VMEM((SB, C2), jnp.int32),
            [pltpu.VMEM((C2, D), jnp.float32)] * 4,
            pltpu.VMEM_SHARED((AROWS, D), jnp.float32),
            [pltpu.SemaphoreType.DMA] * 4,
        ],
    )
    def prop4(t1, t2, t3, t4, srcm_hbm, dstm_hbm, zeros_hbm,
              o1, o2, o3, o4, src_v, dst_v, bufs, acc, gsem):
        c = lax.axis_index("c")
        s = lax.axis_index("s")

        def one(tab, scat_hbm, out_hbm):
            pltpu.sync_copy(zeros_hbm, acc.at[pl.ds(s * ARW, ARW)])
            plsc.subcore_barrier()

            def gath(r, k):
                pltpu.async_copy(tab.at[src_v.at[r]], bufs[k], gsem[k])

            def wait_g(r, k):
                pltpu.make_async_copy(tab.at[src_v.at[r]], bufs[k],
                                      gsem[k]).wait()

            def scat(r, k):
                pltpu.sync_copy(bufs[k], acc.at[dst_v.at[r]], add=True)

            for m in range(NSB):
                pltpu.sync_copy(srcm_hbm.at[s, pl.ds(m * SB, SB)], src_v)
                pltpu.sync_copy(scat_hbm.at[c, s, pl.ds(m * SB, SB)], dst_v)
                for k in range(4):
                    gath(k, k)

                def body(i2, carry):
                    for k in range(4):
                        r = 4 * i2 + k
                        wait_g(r, k)
                        scat(r, k)
                        gath(r + 4, k)
                    return carry

                lax.fori_loop(0, SB // 4 - 1, body, 0)
                for k in range(4):
                    r = SB - 4 + k
                    wait_g(r, k)
                    scat(r, k)

            plsc.subcore_barrier()
            pltpu.sync_copy(acc.at[pl.ds(s * ARW, ARW)], out_hbm.at[c, s])
            plsc.subcore_barrier()

        one(t1, dstm_hbm, o1)
        one(t2, dstm_hbm, o2)
        one(t3, dstm_hbm, o3)
        one(t4, dstm_hbm, o4)

    return prop4


_prop4 = _make_prop4()


def _pad_idx(a, fill):
    npad = IDXROWS * C2 - ECW
    if fill is None:
        pad = jnp.broadcast_to(jnp.arange(npad, dtype=jnp.int32) % N,
                               (a.shape[0], npad))
    elif fill == -1:
        pad = jnp.broadcast_to(
            JBASE + (jnp.arange(npad, dtype=jnp.int32) & JMASK),
            (a.shape[0], npad))
    else:
        pad = jnp.full((a.shape[0], npad), fill, jnp.int32)
    return jnp.concatenate([a, pad], axis=1).reshape(a.shape[0], IDXROWS, C2)


def _local_dst(d):
    j = JBASE + (d & JMASK)
    l0 = jnp.where(d < HN, d, j)
    l1 = jnp.where(d >= HN, d - HN, j)
    return l0, l1


def _prop_indices(idx):
    l0, l1 = _local_dst(idx)
    return jnp.stack([_pad_idx(l0.reshape(NS, ECW), -1),
                      _pad_idx(l1.reshape(NS, ECW), -1)])


def _merge_halves(out):
    halves = out.reshape(NC, AROWS, D)
    return jnp.concatenate([halves[0, :HN], halves[1, :HN]], axis=0)


def _prep_body(x_ref, d_ref, w0_ref, b_ref, xs_ref, xc_ref):
    x = x_ref[0]
    d = d_ref[...]
    xs_ref[0] = x * d
    xc_ref[0] = jnp.dot(x, w0_ref[...], preferred_element_type=jnp.float32) + b_ref[...]


def _tc_prep(x, dis2d, w0stack, bstack):
    return pl.pallas_call(
        _prep_body,
        grid=(T, NB),
        in_specs=[
            pl.BlockSpec((1, BN, D), lambda t, i: (t, i, 0)),
            pl.BlockSpec((BN, 1), lambda t, i: (i, 0)),
            pl.BlockSpec((D, 3 * D), lambda t, i: (0, 0)),
            pl.BlockSpec((1, 3 * D), lambda t, i: (0, 0)),
        ],
        out_specs=[
            pl.BlockSpec((1, BN, D), lambda t, i: (t, i, 0)),
            pl.BlockSpec((1, BN, 3 * D), lambda t, i: (t, i, 0)),
        ],
        out_shape=[
            jax.ShapeDtypeStruct((T, N, D), jnp.float32),
            jax.ShapeDtypeStruct((T, N, 3 * D), jnp.float32),
        ],
    )(x, dis2d, w0stack, bstack)


def _xmerge_body(p_ref, d_ref, xc_ref, w1_ref, out_ref):
    p = p_ref[0]
    d = d_ref[...]
    pxs = -(d * p)
    out_ref[0] = xc_ref[0] + jnp.dot(pxs, w1_ref[...], preferred_element_type=jnp.float32)


def _tc_xmerge(pxp, dis2d, xc, w1stack):
    return pl.pallas_call(
        _xmerge_body,
        grid=(T, NB),
        in_specs=[
            pl.BlockSpec((1, BN, D), lambda t, i: (t, i, 0)),
            pl.BlockSpec((BN, 1), lambda t, i: (i, 0)),
            pl.BlockSpec((1, BN, 3 * D), lambda t, i: (t, i, 0)),
            pl.BlockSpec((D, 3 * D), lambda t, i: (0, 0)),
        ],
        out_specs=pl.BlockSpec((1, BN, 3 * D), lambda t, i: (t, i, 0)),
        out_shape=jax.ShapeDtypeStruct((T, N, 3 * D), jnp.float32),
    )(pxp, dis2d, xc, w1stack)


def _step0_body(xc_ref, h_ref):
    c = xc_ref[...]
    z = jax.nn.sigmoid(c[:, :D])
    ht = jnp.tanh(c[:, 2 * D:])
    h_ref[...] = (1.0 - z) * ht


def _tc_step0(xc0):
    return pl.pallas_call(
        _step0_body,
        grid=(NB,),
        in_specs=[pl.BlockSpec((BN, 3 * D), lambda i: (i, 0))],
        out_specs=pl.BlockSpec((BN, D), lambda i: (i, 0)),
        out_shape=jax.ShapeDtypeStruct((N, D), jnp.float32),
    )(xc0)


def _preh_body(h_ref, d_ref, w_ref, hs_ref, hw_ref):
    h = h_ref[...]
    hs_ref[...] = h * d_ref[...]
    hw_ref[...] = jnp.dot(h, w_ref[...], preferred_element_type=jnp.float32)


def _tc_preh(h, dis2d, wh0zr):
    return pl.pallas_call(
        _preh_body,
        grid=(NB,),
        in_specs=[
            pl.BlockSpec((BN, D), lambda i: (i, 0)),
            pl.BlockSpec((BN, 1), lambda i: (i, 0)),
            pl.BlockSpec((D, 2 * D), lambda i: (0, 0)),
        ],
        out_specs=[
            pl.BlockSpec((BN, D), lambda i: (i, 0)),
            pl.BlockSpec((BN, 2 * D), lambda i: (i, 0)),
        ],
        out_shape=[
            jax.ShapeDtypeStruct((N, D), jnp.float32),
            jax.ShapeDtypeStruct((N, 2 * D), jnp.float32),
        ],
    )(h, dis2d, wh0zr)


def _gates_body(xczr_ref, hw_ref, php_ref, d_ref, w1_ref, h_ref, whh0_ref,
                z_ref, hrs_ref, hrw_ref):
    p = php_ref[...]
    d = d_ref[...]
    phs = -(d * p)
    g = xczr_ref[...] + hw_ref[...] + jnp.dot(phs, w1_ref[...],
                                              preferred_element_type=jnp.float32)
    z = jax.nn.sigmoid(g[:, :D])
    r = jax.nn.sigmoid(g[:, D:])
    hr = h_ref[...] * r
    z_ref[...] = z
    hrs_ref[...] = d * hr
    hrw_ref[...] = jnp.dot(hr, whh0_ref[...], preferred_element_type=jnp.float32)


def _tc_gates(xczr, hw0, php, dis2d, wh1zr, h, whh0):
    return pl.pallas_call(
        _gates_body,
        grid=(NB,),
        in_specs=[
            pl.BlockSpec((BN, 2 * D), lambda i: (i, 0)),
            pl.BlockSpec((BN, 2 * D), lambda i: (i, 0)),
            pl.BlockSpec((BN, D), lambda i: (i, 0)),
            pl.BlockSpec((BN, 1), lambda i: (i, 0)),
            pl.BlockSpec((D, 2 * D), lambda i: (0, 0)),
            pl.BlockSpec((BN, D), lambda i: (i, 0)),
            pl.BlockSpec((D, D), lambda i: (0, 0)),
        ],
        out_specs=[
            pl.BlockSpec((BN, D), lambda i: (i, 0)),
            pl.BlockSpec((BN, D), lambda i: (i, 0)),
            pl.BlockSpec((BN, D), lambda i: (i, 0)),
        ],
        out_shape=[
            jax.ShapeDtypeStruct((N, D), jnp.float32),
            jax.ShapeDtypeStruct((N, D), jnp.float32),
            jax.ShapeDtypeStruct((N, D), jnp.float32),
        ],
    )(xczr, hw0, php, dis2d, wh1zr, h, whh0)


def _newh_body(xch_ref, hrw_ref, pqp_ref, d_ref, whh1_ref, z_ref, h_ref, out_ref):
    q = -(d_ref[...] * pqp_ref[...])
    ht = jnp.tanh(xch_ref[...] + hrw_ref[...] +
                  jnp.dot(q, whh1_ref[...], preferred_element_type=jnp.float32))
    z = z_ref[...]
    out_ref[...] = z * h_ref[...] + (1.0 - z) * ht


def _tc_newh(xch, hrw0, pqp, dis2d, whh1, z, h):
    return pl.pallas_call(
        _newh_body,
        grid=(NB,),
        in_specs=[
            pl.BlockSpec((BN, D), lambda i: (i, 0)),
            pl.BlockSpec((BN, D), lambda i: (i, 0)),
            pl.BlockSpec((BN, D), lambda i: (i, 0)),
            pl.BlockSpec((BN, 1), lambda i: (i, 0)),
            pl.BlockSpec((D, D), lambda i: (0, 0)),
            pl.BlockSpec((BN, D), lambda i: (i, 0)),
            pl.BlockSpec((BN, D), lambda i: (i, 0)),
        ],
        out_specs=pl.BlockSpec((BN, D), lambda i: (i, 0)),
        out_shape=jax.ShapeDtypeStruct((N, D), jnp.float32),
    )(xch, hrw0, pqp, dis2d, whh1, z, h)


def _heads_body(h_ref, wmu_ref, bmu_ref, wsig_ref, bsig_ref, mu_ref, sig_ref):
    h = h_ref[...]
    mu_ref[...] = jnp.dot(h, wmu_ref[...], preferred_element_type=jnp.float32) + bmu_ref[...]
    s = jnp.dot(h, wsig_ref[...], preferred_element_type=jnp.float32) + bsig_ref[...]
    sig_ref[...] = jnp.logaddexp(s, 0.0)


def _tc_heads(h, wmu, bmu, wsig, bsig):
    return pl.pallas_call(
        _heads_body,
        grid=(NB,),
        in_specs=[
            pl.BlockSpec((BN, D), lambda i: (i, 0)),
            pl.BlockSpec((D, 2), lambda i: (0, 0)),
            pl.BlockSpec((1, 2), lambda i: (0, 0)),
            pl.BlockSpec((D, 2), lambda i: (0, 0)),
            pl.BlockSpec((1, 2), lambda i: (0, 0)),
        ],
        out_specs=[
            pl.BlockSpec((BN, 2), lambda i: (i, 0)),
            pl.BlockSpec((BN, 2), lambda i: (i, 0)),
        ],
        out_shape=[
            jax.ShapeDtypeStruct((N, 2), jnp.float32),
            jax.ShapeDtypeStruct((N, 2), jnp.float32),
        ],
    )(h, wmu, bmu[None, :], wsig, bsig[None, :])


def kernel(in_tensor, edge_index, Wxz, bxz, Whz, bhz, Wxr, bxr, Whr, bhr,
           Wxh, bxh, Whh, bhh, Wmu, bmu, Wsig, bsig):
    src = edge_index[0]
    dst = edge_index[1]
    srcm = _pad_idx(src.reshape(NS, ECW), None)
    dstm = _prop_indices(dst)
    srcsc = _prop_indices(src)
    x = in_tensor[0]

    zeros128 = jnp.zeros((ARW, D), jnp.float32)
    ones128 = jnp.ones((N, D), jnp.float32)

    deg = _merge_halves(_prop128(ones128, srcm, srcsc, zeros128))[:, 0]
    dis = jnp.where(deg > 0, lax.rsqrt(jnp.maximum(deg, 1e-12)), 0.0)
    dis2d = dis[:, None]

    w0stack = jnp.concatenate([Wxz[0], Wxr[0], Wxh[0]], axis=1)
    w1stack = jnp.concatenate([Wxz[1], Wxr[1], Wxh[1]], axis=1)
    bstack = jnp.concatenate([bxz + bhz, bxr + bhr, bxh + bhh])[None, :]
    wh0zr = jnp.concatenate([Whz[0], Whr[0]], axis=1)
    wh1zr = jnp.concatenate([Whz[1], Whr[1]], axis=1)

    xs, xc = _tc_prep(x, dis2d, w0stack, bstack)
    pxo = _prop4(xs[0], xs[1], xs[2], xs[3], srcm, dstm, zeros128)
    pxp = jnp.stack([_merge_halves(o) for o in pxo])
    xcf = _tc_xmerge(pxp, dis2d, xc, w1stack)

    h = _tc_step0(xcf[0])
    for t in range(1, T):
        hs, hw0 = _tc_preh(h, dis2d, wh0zr)
        php = _merge_halves(_prop128(hs, srcm, dstm, zeros128))
        z, hrs, hrw0 = _tc_gates(xcf[t, :, :2 * D], hw0, php, dis2d, wh1zr, h, Whh[0])
        pqp = _merge_halves(_prop128(hrs, srcm, dstm, zeros128))
        h = _tc_newh(xcf[t, :, 2 * D:], hrw0, pqp, dis2d, Whh[1], z, h)

    mu, sig = _tc_heads(h, Wmu, bmu, Wsig, bsig)
    return (mu[None], sig[None])
